# baseline (speedup 1.0000x reference)
import functools

import jax
import jax.numpy as jnp
from jax import lax
from jax.experimental import pallas as pl
from jax.experimental.pallas import tpu as pltpu
from jax.experimental.pallas import tpu_sc as plsc

_B, _S, _D = 16, 2048, 64
_N = _B * _S
_NW = 32
_RPW = _N // _NW
_G = 128
_NG = _RPW // _G
_CHUNK = 256
_NCHUNK = _RPW // _CHUNK
_L = 16

_mesh = plsc.VectorSubcoreMesh(core_axis_name="c", subcore_axis_name="s")


@functools.partial(
    pl.kernel,
    mesh=_mesh,
    out_type=jax.ShapeDtypeStruct((_B, _S, _D), jnp.float32),
    scratch_types=[
        pltpu.VMEM((_NG, _G), jnp.int32),
        pltpu.VMEM((_RPW, _D), jnp.float32),
        pltpu.VMEM((_CHUNK, _D), jnp.float32),
        pltpu.VMEM((_CHUNK, _D), jnp.float32),
        pltpu.SemaphoreType.DMA,
        pltpu.SemaphoreType.DMA,
    ],
    compiler_params=pltpu.CompilerParams(use_tc_tiling_on_sc=False),
)
def _emb_lookup(x_hbm, tok_hbm, pos_hbm, out_hbm, ids_v, rows_v, pb0, pb1,
                gsem, ssem):
    cid = lax.axis_index("c")
    sid = lax.axis_index("s")
    wid = sid * 2 + cid
    base = wid * _RPW
    pos_base = lax.rem(base, _S)

    x0 = pl.multiple_of(wid * _NG, 8)
    pltpu.sync_copy(x_hbm.at[pl.ds(x0, _NG)], ids_v)

    gcps = [
        pltpu.async_copy(
            tok_hbm.at[ids_v.at[g]],
            rows_v.at[pl.ds(g * _G, _G)],
            gsem,
        )
        for g in range(_NG)
    ]

    pbs = [pb0, pb1]
    scps = [None] * _NCHUNK
    gpc = _CHUNK // _G
    for k in range(_NCHUNK):
        pb = pbs[k % 2]
        if k >= 2:
            scps[k - 2].wait()
        p0 = pl.multiple_of(pos_base + k * _CHUNK, 8)
        pltpu.sync_copy(pos_hbm.at[pl.ds(p0, _CHUNK)], pb)
        for g in range(gpc):
            gcps[k * gpc + g].wait()

        @plsc.parallel_loop(0, _CHUNK, unroll=8)
        def _add(r):
            for c in range(_D // _L):
                sl = pl.ds(c * _L, _L)
                pb[r, sl] = pb[r, sl] + rows_v[k * _CHUNK + r, sl]

        b = base // _S
        s0 = pl.multiple_of(pos_base + k * _CHUNK, 8)
        scps[k] = pltpu.async_copy(pb, out_hbm.at[b, pl.ds(s0, _CHUNK)],
                                   ssem)
    scps[_NCHUNK - 2].wait()
    scps[_NCHUNK - 1].wait()


def kernel(x, token_emb, pos_emb):
    idx = x.astype(jnp.int32).reshape(_N // _G, _G)
    return _emb_lookup(idx, token_emb, pos_emb)
